# baseline (speedup 1.0000x reference)
import functools

import jax
import jax.numpy as jnp
from jax import lax
from jax.experimental import pallas as pl
from jax.experimental.pallas import tpu as pltpu
from jax.experimental.pallas import tpu_sc as plsc

_H = 128
_N = 10000
_E = 320000
_W = 144
_SUB = 50
_NC = 1
_NS = 16
_NW = _NC * _NS
_EPT = _E // _NW
_JPT = _EPT // _SUB
_NOCT = _JPT // 8
_RPS = 624
_TAIL = _N - _NS * _RPS


def _prologue_body(x_ref, wg1_ref, bg1_ref, wg2_ref, bg2_ref, wt_ref,
                   bt_ref, r_ref):
    x = x_ref[...]
    h = jnp.maximum(
        jnp.dot(x, wg1_ref[...], preferred_element_type=jnp.float32)
        + bg1_ref[...], 0.0)
    g = (jnp.dot(h, wg2_ref[...], preferred_element_type=jnp.float32)
         + bg2_ref[...])
    p = jnp.exp(g - jnp.max(g))
    t = jnp.maximum(
        jnp.dot(x, wt_ref[...], preferred_element_type=jnp.float32)
        + bt_ref[...], 0.0)
    r_ref[:, :_H] = p * t
    r_ref[:, _H:_H + 1] = p
    r_ref[:, _H + 1:] = jnp.zeros((x.shape[0], _W - _H - 1), jnp.float32)


def _edge_body(r_hbm, idx_hbm, z2_hbm, out_hbm,
               idx0, idx1, idx2, idx3, idx4, idx5, idx6, idx7,
               rows0, rows1, rows2, rows3, accum,
               isem0, isem1, isem2, isem3, isem4, isem5, isem6, isem7,
               gsem0, gsem1, gsem2, gsem3, ssem0, ssem1, ssem2, ssem3):
    c = lax.axis_index("c")
    s = lax.axis_index("s")
    wid = c * _NS + s
    base = wid * _JPT
    ibufs = ((idx0, isem0), (idx1, isem1), (idx2, isem2), (idx3, isem3),
             (idx4, isem4), (idx5, isem5), (idx6, isem6), (idx7, isem7))
    rbufs = ((rows0, gsem0, ssem0), (rows1, gsem1, ssem1),
             (rows2, gsem2, ssem2), (rows3, gsem3, ssem3))
    if True:
        pltpu.sync_copy(z2_hbm.at[pl.ds(s * _RPS, _RPS)],
                        accum.at[pl.ds(s * _RPS, _RPS)])

        @pl.when(s == 0)
        def _():
            pltpu.sync_copy(z2_hbm.at[pl.ds(_NS * _RPS, _TAIL)],
                            accum.at[pl.ds(_NS * _RPS, _TAIL)])

        plsc.subcore_barrier()

        for jj in range(4):
            ib, isem = ibufs[jj]
            pltpu.async_copy(idx_hbm.at[base + jj], ib, isem)
        for jj in range(2):
            ib, isem = ibufs[jj]
            rows, gsem, _ = rbufs[jj]
            pltpu.make_async_copy(idx_hbm.at[base + jj], ib, isem).wait()
            pltpu.async_copy(r_hbm.at[ib.at[0]], rows, gsem)

        def step(j, m):
            rows, gsem, ssem = rbufs[m % 4]
            ib, _ = ibufs[m % 8]
            rowsN, gsemN, ssemN = rbufs[(m + 2) % 4]
            ibN, isemN = ibufs[(m + 2) % 8]
            ibL, isemL = ibufs[(m + 4) % 8]
            pltpu.make_async_copy(r_hbm.at[ib.at[0]], rows, gsem).wait()
            pltpu.async_copy(rows, accum.at[ib.at[1]], ssem, add=True)

            @pl.when(j >= 2)
            def _():
                pltpu.make_async_copy(
                    rowsN, accum.at[ibN.at[1]], ssemN).wait()

            @pl.when(j + 2 < _JPT)
            def _():
                pltpu.make_async_copy(
                    idx_hbm.at[base + j + 2], ibN, isemN).wait()
                pltpu.async_copy(r_hbm.at[ibN.at[0]], rowsN, gsemN)

            @pl.when(j + 4 < _JPT)
            def _():
                pltpu.async_copy(idx_hbm.at[base + j + 4], ibL, isemL)

        def body(k, carry):
            for m in range(8):
                step(8 * k + m, m)
            return carry

        lax.fori_loop(0, _NOCT, body, 0)
        pltpu.make_async_copy(
            rows2, accum.at[idx6.at[1]], ssem2).wait()
        pltpu.make_async_copy(
            rows3, accum.at[idx7.at[1]], ssem3).wait()
        plsc.subcore_barrier()
        pltpu.sync_copy(accum.at[pl.ds(s * _RPS, _RPS)],
                        out_hbm.at[c].at[pl.ds(s * _RPS, _RPS)])

        @pl.when(s == 0)
        def _():
            pltpu.sync_copy(accum.at[pl.ds(_NS * _RPS, _TAIL)],
                            out_hbm.at[c].at[pl.ds(_NS * _RPS, _TAIL)])


def _epilogue_body(u_ref, z_ref):
    u = u_ref[0]
    z_ref[...] = u[:, :_H] / (u[:, _H:_H + 1] + 1e-16)


def kernel(metabolite_features, hyperedge_index, W_g1, b_g1, W_g2, b_g2,
           W_t, b_t):
    x = metabolite_features
    idx = jnp.stack(
        [hyperedge_index[0].reshape(_NW, _JPT, _SUB),
         hyperedge_index[1].reshape(_NW, _JPT, _SUB)],
        axis=2).reshape(_NW * _JPT, 2, _SUB)

    r = pl.pallas_call(
        _prologue_body,
        out_shape=jax.ShapeDtypeStruct((_N, _W), jnp.float32),
    )(x, W_g1, b_g1.reshape(1, -1), W_g2, b_g2.reshape(1, -1),
      W_t, b_t.reshape(1, -1))

    mesh = plsc.VectorSubcoreMesh(core_axis_name="c", subcore_axis_name="s", num_cores=1)
    edge_kernel = functools.partial(
        pl.kernel,
        mesh=mesh,
        out_type=jax.ShapeDtypeStruct((_NC, _N, _W), jnp.float32),
        scratch_types=(
            [pltpu.VMEM((2, _SUB), jnp.int32) for _ in range(8)]
            + [pltpu.VMEM((_SUB, _W), jnp.float32) for _ in range(4)]
            + [pltpu.VMEM_SHARED((_N, _W), jnp.float32)]
            + [pltpu.SemaphoreType.DMA for _ in range(16)]
        ),
        compiler_params=pltpu.CompilerParams(
            needs_layout_passes=False, use_tc_tiling_on_sc=False),
    )(_edge_body)
    u = edge_kernel(r, idx, jnp.zeros((_N, _W), jnp.float32))

    z = pl.pallas_call(
        _epilogue_body,
        out_shape=jax.ShapeDtypeStruct((_N, _H), jnp.float32),
    )(u)
    return z


# speedup vs baseline: 1.3609x; 1.3609x over previous
import functools

import jax
import jax.numpy as jnp
from jax import lax
from jax.experimental import pallas as pl
from jax.experimental.pallas import tpu as pltpu
from jax.experimental.pallas import tpu_sc as plsc

_H = 128
_N = 10000
_E = 320000
_W = 144
_SUB = 50
_NC = 2
_NS = 16
_NW = _NC * _NS
_EPT = _E // _NW
_JPT = _EPT // _SUB
_NOCT = _JPT // 8
_RPS = 624
_TAIL = _N - _NS * _RPS


def _prologue_body(x_ref, wg1_ref, bg1_ref, wg2_ref, bg2_ref, wt_ref,
                   bt_ref, r_ref):
    x = x_ref[...]
    h = jnp.maximum(
        jnp.dot(x, wg1_ref[...], preferred_element_type=jnp.float32)
        + bg1_ref[...], 0.0)
    g = (jnp.dot(h, wg2_ref[...], preferred_element_type=jnp.float32)
         + bg2_ref[...])
    p = jnp.exp(g - jnp.max(g))
    t = jnp.maximum(
        jnp.dot(x, wt_ref[...], preferred_element_type=jnp.float32)
        + bt_ref[...], 0.0)
    r_ref[:, :_H] = p * t
    r_ref[:, _H:_H + 1] = p
    r_ref[:, _H + 1:] = jnp.zeros((x.shape[0], _W - _H - 1), jnp.float32)


def _edge_body(r_hbm, idx_hbm, z2_hbm, out0_hbm, out1_hbm,
               idx0, idx1, idx2, idx3, idx4, idx5, idx6, idx7,
               rows0, rows1, rows2, rows3, accum,
               isem0, isem1, isem2, isem3, isem4, isem5, isem6, isem7,
               gsem0, gsem1, gsem2, gsem3, ssem0, ssem1, ssem2, ssem3):
    c = lax.axis_index("c")
    s = lax.axis_index("s")
    wid = c * _NS + s
    base = wid * _JPT
    ibufs = ((idx0, isem0), (idx1, isem1), (idx2, isem2), (idx3, isem3),
             (idx4, isem4), (idx5, isem5), (idx6, isem6), (idx7, isem7))
    rbufs = ((rows0, gsem0, ssem0), (rows1, gsem1, ssem1),
             (rows2, gsem2, ssem2), (rows3, gsem3, ssem3))
    if True:
        pltpu.sync_copy(z2_hbm.at[pl.ds(s * _RPS, _RPS)],
                        accum.at[pl.ds(s * _RPS, _RPS)])

        @pl.when(s == 0)
        def _():
            pltpu.sync_copy(z2_hbm.at[pl.ds(_NS * _RPS, _TAIL)],
                            accum.at[pl.ds(_NS * _RPS, _TAIL)])

        plsc.subcore_barrier()

        for jj in range(4):
            ib, isem = ibufs[jj]
            pltpu.async_copy(idx_hbm.at[base + jj], ib, isem)
        for jj in range(2):
            ib, isem = ibufs[jj]
            rows, gsem, _ = rbufs[jj]
            pltpu.make_async_copy(idx_hbm.at[base + jj], ib, isem).wait()
            pltpu.async_copy(r_hbm.at[ib.at[0]], rows, gsem)

        def step(j, m):
            rows, gsem, ssem = rbufs[m % 4]
            ib, _ = ibufs[m % 8]
            rowsN, gsemN, ssemN = rbufs[(m + 2) % 4]
            ibN, isemN = ibufs[(m + 2) % 8]
            ibL, isemL = ibufs[(m + 4) % 8]
            pltpu.make_async_copy(r_hbm.at[ib.at[0]], rows, gsem).wait()
            pltpu.async_copy(rows, accum.at[ib.at[1]], ssem, add=True)

            @pl.when(j >= 2)
            def _():
                pltpu.make_async_copy(
                    rowsN, accum.at[ibN.at[1]], ssemN).wait()

            @pl.when(j + 2 < _JPT)
            def _():
                pltpu.make_async_copy(
                    idx_hbm.at[base + j + 2], ibN, isemN).wait()
                pltpu.async_copy(r_hbm.at[ibN.at[0]], rowsN, gsemN)

            @pl.when(j + 4 < _JPT)
            def _():
                pltpu.async_copy(idx_hbm.at[base + j + 4], ibL, isemL)

        def body(k, carry):
            for m in range(8):
                step(8 * k + m, m)
            return carry

        lax.fori_loop(0, _NOCT, body, 0)
        pltpu.make_async_copy(
            rows2, accum.at[idx6.at[1]], ssem2).wait()
        pltpu.make_async_copy(
            rows3, accum.at[idx7.at[1]], ssem3).wait()
        plsc.subcore_barrier()

        @pl.when(c == 0)
        def _():
            pltpu.sync_copy(accum.at[pl.ds(s * _RPS, _RPS)],
                            out0_hbm.at[pl.ds(s * _RPS, _RPS)])

            @pl.when(s == 0)
            def _():
                pltpu.sync_copy(accum.at[pl.ds(_NS * _RPS, _TAIL)],
                                out0_hbm.at[pl.ds(_NS * _RPS, _TAIL)])

        @pl.when(c == 1)
        def _():
            pltpu.sync_copy(accum.at[pl.ds(s * _RPS, _RPS)],
                            out1_hbm.at[pl.ds(s * _RPS, _RPS)])

            @pl.when(s == 0)
            def _():
                pltpu.sync_copy(accum.at[pl.ds(_NS * _RPS, _TAIL)],
                                out1_hbm.at[pl.ds(_NS * _RPS, _TAIL)])


def _epilogue_body(u0_ref, u1_ref, z_ref):
    u = u0_ref[...] + u1_ref[...]
    z_ref[...] = u[:, :_H] / (u[:, _H:_H + 1] + 1e-16)


def kernel(metabolite_features, hyperedge_index, W_g1, b_g1, W_g2, b_g2,
           W_t, b_t):
    x = metabolite_features
    idx = jnp.stack(
        [hyperedge_index[0].reshape(_NW, _JPT, _SUB),
         hyperedge_index[1].reshape(_NW, _JPT, _SUB)],
        axis=2).reshape(_NW * _JPT, 2, _SUB)

    r = pl.pallas_call(
        _prologue_body,
        out_shape=jax.ShapeDtypeStruct((_N, _W), jnp.float32),
    )(x, W_g1, b_g1.reshape(1, -1), W_g2, b_g2.reshape(1, -1),
      W_t, b_t.reshape(1, -1))

    mesh = plsc.VectorSubcoreMesh(core_axis_name="c", subcore_axis_name="s")
    edge_kernel = functools.partial(
        pl.kernel,
        mesh=mesh,
        out_type=(jax.ShapeDtypeStruct((_N, _W), jnp.float32),
                  jax.ShapeDtypeStruct((_N, _W), jnp.float32)),
        scratch_types=(
            [pltpu.VMEM((2, _SUB), jnp.int32) for _ in range(8)]
            + [pltpu.VMEM((_SUB, _W), jnp.float32) for _ in range(4)]
            + [pltpu.VMEM_SHARED((_N, _W), jnp.float32)]
            + [pltpu.SemaphoreType.DMA for _ in range(16)]
        ),
        compiler_params=pltpu.CompilerParams(
            needs_layout_passes=False, use_tc_tiling_on_sc=False),
    )(_edge_body)
    u0, u1 = edge_kernel(r, idx, jnp.zeros((_N, _W), jnp.float32))

    z = pl.pallas_call(
        _epilogue_body,
        out_shape=jax.ShapeDtypeStruct((_N, _H), jnp.float32),
    )(u0, u1)
    return z


# speedup vs baseline: 1.6255x; 1.1944x over previous
import functools

import jax
import jax.numpy as jnp
from jax import lax
from jax.experimental import pallas as pl
from jax.experimental.pallas import tpu as pltpu
from jax.experimental.pallas import tpu_sc as plsc

_H = 128
_N = 10000
_E = 320000
_W = 144
_SUB = 100
_NC = 2
_NS = 16
_NW = _NC * _NS
_EPT = _E // _NW
_JPT = _EPT // _SUB
_NQ = _JPT // 4
_RPS = 624
_TAIL = _N - _NS * _RPS


def _prologue_body(x_ref, wg1_ref, bg1_ref, wg2_ref, bg2_ref, wt_ref,
                   bt_ref, r_ref):
    x = x_ref[...]
    h = jnp.maximum(
        jnp.dot(x, wg1_ref[...], preferred_element_type=jnp.float32)
        + bg1_ref[...], 0.0)
    g = (jnp.dot(h, wg2_ref[...], preferred_element_type=jnp.float32)
         + bg2_ref[...])
    p = jnp.exp(g - jnp.max(g))
    t = jnp.maximum(
        jnp.dot(x, wt_ref[...], preferred_element_type=jnp.float32)
        + bt_ref[...], 0.0)
    r_ref[:, :_H] = p * t
    r_ref[:, _H:_H + 1] = p
    r_ref[:, _H + 1:] = jnp.zeros((x.shape[0], _W - _H - 1), jnp.float32)


def _edge_body(r_hbm, idx_hbm, z2_hbm, out_hbm,
               idx0, idx1, idx2, idx3,
               rows0, rows1, accum,
               isem0, isem1, isem2, isem3,
               gsem0, gsem1, ssem0, ssem1):
    c = lax.axis_index("c")
    s = lax.axis_index("s")
    wid = c * _NS + s
    base = wid * _JPT
    ibufs = ((idx0, isem0), (idx1, isem1), (idx2, isem2), (idx3, isem3))
    rbufs = ((rows0, gsem0, ssem0), (rows1, gsem1, ssem1))
    if True:
        pltpu.sync_copy(z2_hbm.at[pl.ds(s * _RPS, _RPS)],
                        accum.at[pl.ds(s * _RPS, _RPS)])

        @pl.when(s == 0)
        def _():
            pltpu.sync_copy(z2_hbm.at[pl.ds(_NS * _RPS, _TAIL)],
                            accum.at[pl.ds(_NS * _RPS, _TAIL)])

        plsc.subcore_barrier()

        for jj in range(4):
            ib, isem = ibufs[jj]
            pltpu.async_copy(idx_hbm.at[base + jj], ib, isem)
        for jj in range(2):
            ib, isem = ibufs[jj]
            rows, gsem, _ = rbufs[jj]
            pltpu.make_async_copy(idx_hbm.at[base + jj], ib, isem).wait()
            pltpu.async_copy(r_hbm.at[ib.at[0]], rows, gsem)

        def step(j, m):
            rows, gsem, ssem = rbufs[m % 2]
            ib, isem = ibufs[m % 4]
            ibN, isemN = ibufs[(m + 2) % 4]
            pltpu.make_async_copy(r_hbm.at[ib.at[0]], rows, gsem).wait()
            pltpu.async_copy(rows, accum.at[ib.at[1]], ssem,
                             add=True).wait()

            @pl.when(j + 2 < _JPT)
            def _():
                pltpu.make_async_copy(
                    idx_hbm.at[base + j + 2], ibN, isemN).wait()
                pltpu.async_copy(r_hbm.at[ibN.at[0]], rows, gsem)

            @pl.when(j + 4 < _JPT)
            def _():
                pltpu.async_copy(idx_hbm.at[base + j + 4], ib, isem)

        def body(k, carry):
            for m in range(4):
                step(4 * k + m, m)
            return carry

        lax.fori_loop(0, _NQ, body, 0)
        plsc.subcore_barrier()
        pltpu.sync_copy(accum.at[pl.ds(s * _RPS, _RPS)],
                        out_hbm.at[c].at[pl.ds(s * _RPS, _RPS)])

        @pl.when(s == 0)
        def _():
            pltpu.sync_copy(accum.at[pl.ds(_NS * _RPS, _TAIL)],
                            out_hbm.at[c].at[pl.ds(_NS * _RPS, _TAIL)])


def _epilogue_body(u_ref, z_ref):
    u = u_ref[0] + u_ref[1]
    z_ref[...] = u[:, :_H] / (u[:, _H:_H + 1] + 1e-16)


def kernel(metabolite_features, hyperedge_index, W_g1, b_g1, W_g2, b_g2,
           W_t, b_t):
    x = metabolite_features
    idx = jnp.stack(
        [hyperedge_index[0].reshape(_NW, _JPT, _SUB),
         hyperedge_index[1].reshape(_NW, _JPT, _SUB)],
        axis=2).reshape(_NW * _JPT, 2, _SUB)

    r = pl.pallas_call(
        _prologue_body,
        out_shape=jax.ShapeDtypeStruct((_N, _W), jnp.float32),
    )(x, W_g1, b_g1.reshape(1, -1), W_g2, b_g2.reshape(1, -1),
      W_t, b_t.reshape(1, -1))

    mesh = plsc.VectorSubcoreMesh(core_axis_name="c", subcore_axis_name="s")
    edge_kernel = functools.partial(
        pl.kernel,
        mesh=mesh,
        out_type=jax.ShapeDtypeStruct((_NC, _N, _W), jnp.float32),
        scratch_types=(
            [pltpu.VMEM((2, _SUB), jnp.int32) for _ in range(4)]
            + [pltpu.VMEM((_SUB, _W), jnp.float32) for _ in range(2)]
            + [pltpu.VMEM_SHARED((_N, _W), jnp.float32)]
            + [pltpu.SemaphoreType.DMA for _ in range(8)]
        ),
        compiler_params=pltpu.CompilerParams(
            needs_layout_passes=False, use_tc_tiling_on_sc=False),
    )(_edge_body)
    u = edge_kernel(r, idx, jnp.zeros((_N, _W), jnp.float32))

    z = pl.pallas_call(
        _epilogue_body,
        out_shape=jax.ShapeDtypeStruct((_N, _H), jnp.float32),
    )(u)
    return z


# speedup vs baseline: 1.7181x; 1.0570x over previous
import functools

import jax
import jax.numpy as jnp
from jax import lax
from jax.experimental import pallas as pl
from jax.experimental.pallas import tpu as pltpu
from jax.experimental.pallas import tpu_sc as plsc

_H = 128
_N = 10000
_E = 320000
_W = 144
_SUB = 125
_NC = 2
_NS = 16
_NW = _NC * _NS
_EPT = _E // _NW
_JPT = _EPT // _SUB
_NQ = _JPT // 4
_RPS = 624
_TAIL = _N - _NS * _RPS


def _prologue_body(x_ref, wg1_ref, bg1_ref, wg2_ref, bg2_ref, wt_ref,
                   bt_ref, r_ref):
    x = x_ref[...]
    h = jnp.maximum(
        jnp.dot(x, wg1_ref[...], preferred_element_type=jnp.float32)
        + bg1_ref[...], 0.0)
    g = (jnp.dot(h, wg2_ref[...], preferred_element_type=jnp.float32)
         + bg2_ref[...])
    p = jnp.exp(g - jnp.max(g))
    t = jnp.maximum(
        jnp.dot(x, wt_ref[...], preferred_element_type=jnp.float32)
        + bt_ref[...], 0.0)
    r_ref[:, :_H] = p * t
    r_ref[:, _H:_H + 1] = p
    r_ref[:, _H + 1:] = jnp.zeros((x.shape[0], _W - _H - 1), jnp.float32)


def _edge_body(r_hbm, idx_hbm, z2_hbm, out_hbm,
               idx0, idx1, idx2, idx3,
               rows0, rows1, accum,
               isem0, isem1, isem2, isem3,
               gsem0, gsem1, ssem0, ssem1):
    c = lax.axis_index("c")
    s = lax.axis_index("s")
    wid = c * _NS + s
    base = wid * _JPT
    ibufs = ((idx0, isem0), (idx1, isem1), (idx2, isem2), (idx3, isem3))
    rbufs = ((rows0, gsem0, ssem0), (rows1, gsem1, ssem1))
    if True:
        pltpu.sync_copy(z2_hbm.at[pl.ds(s * _RPS, _RPS)],
                        accum.at[pl.ds(s * _RPS, _RPS)])

        @pl.when(s == 0)
        def _():
            pltpu.sync_copy(z2_hbm.at[pl.ds(_NS * _RPS, _TAIL)],
                            accum.at[pl.ds(_NS * _RPS, _TAIL)])

        plsc.subcore_barrier()

        for jj in range(4):
            ib, isem = ibufs[jj]
            pltpu.async_copy(idx_hbm.at[base + jj], ib, isem)
        for jj in range(2):
            ib, isem = ibufs[jj]
            rows, gsem, _ = rbufs[jj]
            pltpu.make_async_copy(idx_hbm.at[base + jj], ib, isem).wait()
            pltpu.async_copy(r_hbm.at[ib.at[0]], rows, gsem)

        def step(j, m):
            rows, gsem, ssem = rbufs[m % 2]
            ib, isem = ibufs[m % 4]
            ibN, isemN = ibufs[(m + 2) % 4]
            pltpu.make_async_copy(r_hbm.at[ib.at[0]], rows, gsem).wait()
            pltpu.async_copy(rows, accum.at[ib.at[1]], ssem,
                             add=True).wait()

            @pl.when(j + 2 < _JPT)
            def _():
                pltpu.make_async_copy(
                    idx_hbm.at[base + j + 2], ibN, isemN).wait()
                pltpu.async_copy(r_hbm.at[ibN.at[0]], rows, gsem)

            @pl.when(j + 4 < _JPT)
            def _():
                pltpu.async_copy(idx_hbm.at[base + j + 4], ib, isem)

        def body(k, carry):
            for m in range(4):
                step(4 * k + m, m)
            return carry

        lax.fori_loop(0, _NQ, body, 0)
        plsc.subcore_barrier()
        pltpu.sync_copy(accum.at[pl.ds(s * _RPS, _RPS)],
                        out_hbm.at[c].at[pl.ds(s * _RPS, _RPS)])

        @pl.when(s == 0)
        def _():
            pltpu.sync_copy(accum.at[pl.ds(_NS * _RPS, _TAIL)],
                            out_hbm.at[c].at[pl.ds(_NS * _RPS, _TAIL)])


def _epilogue_body(u_ref, z_ref):
    u = u_ref[0] + u_ref[1]
    z_ref[...] = u[:, :_H] / (u[:, _H:_H + 1] + 1e-16)


def kernel(metabolite_features, hyperedge_index, W_g1, b_g1, W_g2, b_g2,
           W_t, b_t):
    x = metabolite_features
    idx = jnp.stack(
        [hyperedge_index[0].reshape(_NW, _JPT, _SUB),
         hyperedge_index[1].reshape(_NW, _JPT, _SUB)],
        axis=2).reshape(_NW * _JPT, 2, _SUB)

    r = pl.pallas_call(
        _prologue_body,
        out_shape=jax.ShapeDtypeStruct((_N, _W), jnp.float32),
    )(x, W_g1, b_g1.reshape(1, -1), W_g2, b_g2.reshape(1, -1),
      W_t, b_t.reshape(1, -1))

    mesh = plsc.VectorSubcoreMesh(core_axis_name="c", subcore_axis_name="s")
    edge_kernel = functools.partial(
        pl.kernel,
        mesh=mesh,
        out_type=jax.ShapeDtypeStruct((_NC, _N, _W), jnp.float32),
        scratch_types=(
            [pltpu.VMEM((2, _SUB), jnp.int32) for _ in range(4)]
            + [pltpu.VMEM((_SUB, _W), jnp.float32) for _ in range(2)]
            + [pltpu.VMEM_SHARED((_N, _W), jnp.float32)]
            + [pltpu.SemaphoreType.DMA for _ in range(8)]
        ),
        compiler_params=pltpu.CompilerParams(
            needs_layout_passes=False, use_tc_tiling_on_sc=False),
    )(_edge_body)
    u = edge_kernel(r, idx, jnp.zeros((_N, _W), jnp.float32))

    z = pl.pallas_call(
        _epilogue_body,
        out_shape=jax.ShapeDtypeStruct((_N, _H), jnp.float32),
    )(u)
    return z
